# baseline (speedup 1.0000x reference)
import jax
import jax.numpy as jnp
from jax.experimental import pallas as pl
from jax.experimental.pallas import tpu as pltpu


def _make_body(C, H, W, n1, n2, n3):
    HW = H * W
    s1a, p1a = 2, 6
    Ho1a = (H + 2 * p1a - 1) // s1a + 1
    Wo1a = (W + 2 * p1a - 1) // s1a + 1
    i_lo = -(-p1a // s1a)
    j_lo = -(-p1a // s1a)
    nr = (H + 1) // 2
    nc = (W + 1) // 2
    nS = nr * nc
    kh1, kw1, ph1, pw1 = 10, 10, 5, 6
    Hp1, Wp1 = Ho1a + 2 * ph1, Wo1a + 2 * pw1
    Lacc1 = (H - 1) * Wp1 + W
    N1 = 512
    Lb1 = (kh1 - 1) * Wp1 + N1
    kh2, kw2, ph2, pw2 = 3, 3, 1, 1
    Hp2, Wp2 = H + 2 * ph2, W + 2 * pw2
    Lacc2 = (H - 1) * Wp2 + W
    N2 = 384
    Lb2 = (kh2 - 1) * Wp2 + N2

    def body(x_ref, xs_ref, w12a_ref, b12a_ref, w1c_ref, b1b_ref,
             w2c_ref, b2b_ref, w3_ref, b3_ref, o_ref,
             zb1_ref, zr1_ref, zb2_ref, zr2_ref):
        f32, bf16 = jnp.float32, jnp.bfloat16
        x = x_ref[0]
        xs = xs_ref[0]

        m = jnp.max(x, axis=1, keepdims=True)
        x2 = jnp.concatenate([jnp.broadcast_to(m, (C, HW)), x], axis=0)
        x2s = jnp.concatenate([jnp.broadcast_to(m, (C, nS)), xs], axis=0)
        xin = jnp.concatenate([x2, x2s], axis=1).astype(bf16)
        b12a = b12a_ref[...]
        y12 = jnp.dot(w12a_ref[...], xin, preferred_element_type=f32) + b12a
        y12 = jnp.maximum(y12, 0.0)
        y2a = y12[n1:, :HW]
        y1as = y12[:n1, HW:]

        zb1_ref[...] = jnp.zeros_like(zb1_ref)
        rb1a = jnp.maximum(b12a[:n1], 0.0).astype(bf16)
        for r in range(Ho1a):
            st = (ph1 + r) * Wp1 + pw1
            zb1_ref[:, st:st + Wo1a] = jnp.broadcast_to(rb1a, (n1, Wo1a))
        for r in range(nr):
            st = (ph1 + i_lo + r) * Wp1 + (pw1 + j_lo)
            zb1_ref[:, st:st + nc] = y1as[:, r * nc:(r + 1) * nc].astype(bf16)

        for di in range(kh1):
            zr1_ref[di * n1:(di + 1) * n1, :] = \
                zb1_ref[:, di * Wp1:di * Wp1 + N1]

        acc1 = jnp.broadcast_to(b1b_ref[...], (n1, Lacc1)).astype(f32)
        for dj in range(kw1):
            p = jnp.dot(w1c_ref[dj * n1:(dj + 1) * n1], zr1_ref[...],
                        preferred_element_type=f32)
            acc1 = acc1 + p[:, dj:dj + Lacc1]
        h1 = jnp.maximum(acc1, 0.0)

        zb2_ref[...] = jnp.zeros_like(zb2_ref)
        for r in range(H):
            st = (ph2 + r) * Wp2 + pw2
            zb2_ref[:, st:st + W] = y2a[:, r * W:(r + 1) * W].astype(bf16)
        for di in range(kh2):
            zr2_ref[di * n2:(di + 1) * n2, :] = \
                zb2_ref[:, di * Wp2:di * Wp2 + N2]

        p2 = jnp.dot(w2c_ref[...], zr2_ref[...],
                     preferred_element_type=f32)
        acc2 = jnp.broadcast_to(b2b_ref[...], (n2, Lacc2)).astype(f32)
        for dj in range(kw2):
            acc2 = acc2 + p2[dj * n2:(dj + 1) * n2, dj:dj + Lacc2]
        h2 = jnp.maximum(acc2, 0.0)

        h1v = jnp.concatenate(
            [h1[:, r * Wp1:r * Wp1 + W] for r in range(H)], axis=1)
        h2v = jnp.concatenate(
            [h2[:, r * Wp2:r * Wp2 + W] for r in range(H)], axis=1)
        cat = jnp.concatenate([h1v, h2v], axis=0).astype(bf16)
        h3 = jnp.dot(w3_ref[...], cat, preferred_element_type=f32)
        h3 = h3 + b3_ref[...]
        o_ref[0, :n3] = jnp.maximum(h3, 0.0)
        o_ref[0, n3:] = x

    geom = dict(HW=HW, nS=nS, Lb1=Lb1, N1=N1, Lb2=Lb2, N2=N2,
                kh1=kh1, kw1=kw1, kh2=kh2, kw2=kw2)
    return body, geom


def kernel(x, w1a, b1a, w1b, b1b, w2a, b2a, w2b, b2b, w3, b3):
    B, C, H, W = x.shape
    n1, n2, n3 = w1b.shape[0], w2b.shape[0], w3.shape[0]
    body, g = _make_body(C, H, W, n1, n2, n3)
    HW, nS = g["HW"], g["nS"]
    bf16, f32 = jnp.bfloat16, jnp.float32

    x_cm = x.reshape(B, C, HW)
    xs = x[:, :, ::2, ::2].reshape(B, C, nS)
    w12a = jnp.concatenate([w1a.reshape(n1, 2 * C),
                            w2a.reshape(n2, 2 * C)], axis=0).astype(bf16)
    b12a = jnp.concatenate([b1a, b2a]).reshape(n1 + n2, 1).astype(f32)
    w1t = w1b.transpose(2, 3, 0, 1)
    w1c = jnp.concatenate(
        [jnp.concatenate([w1t[di, dj] for di in range(g["kh1"])], axis=1)
         for dj in range(g["kw1"])], axis=0).astype(bf16)
    w2t = w2b.transpose(2, 3, 0, 1)
    w2c = jnp.concatenate(
        [jnp.concatenate([w2t[di, dj] for di in range(g["kh2"])], axis=1)
         for dj in range(g["kw2"])], axis=0).astype(bf16)
    w3m = w3.reshape(n3, n1 + n2).astype(bf16)
    b1bm = b1b.reshape(n1, 1).astype(f32)
    b2bm = b2b.reshape(n2, 1).astype(f32)
    b3m = b3.reshape(n3, 1).astype(f32)

    out = pl.pallas_call(
        body,
        out_shape=jax.ShapeDtypeStruct((B, n3 + C, HW), f32),
        grid=(B,),
        in_specs=[
            pl.BlockSpec((1, C, HW), lambda b: (b, 0, 0)),
            pl.BlockSpec((1, C, nS), lambda b: (b, 0, 0)),
            pl.BlockSpec((n1 + n2, 2 * C), lambda b: (0, 0)),
            pl.BlockSpec((n1 + n2, 1), lambda b: (0, 0)),
            pl.BlockSpec((g["kw1"] * n1, g["kh1"] * n1), lambda b: (0, 0)),
            pl.BlockSpec((n1, 1), lambda b: (0, 0)),
            pl.BlockSpec((g["kw2"] * n2, g["kh2"] * n2), lambda b: (0, 0)),
            pl.BlockSpec((n2, 1), lambda b: (0, 0)),
            pl.BlockSpec((n3, n1 + n2), lambda b: (0, 0)),
            pl.BlockSpec((n3, 1), lambda b: (0, 0)),
        ],
        out_specs=pl.BlockSpec((1, n3 + C, HW), lambda b: (b, 0, 0)),
        scratch_shapes=[pltpu.VMEM((n1, g["Lb1"]), bf16),
                        pltpu.VMEM((g["kh1"] * n1, g["N1"]), bf16),
                        pltpu.VMEM((n2, g["Lb2"]), bf16),
                        pltpu.VMEM((g["kh2"] * n2, g["N2"]), bf16)],
        compiler_params=pltpu.CompilerParams(dimension_semantics=("parallel",)),
    )(x_cm, xs, w12a, b12a, w1c, b1bm, w2c, b2bm, w3m, b3m)
    return out.reshape(B, n3 + C, H, W)


# speedup vs baseline: 1.0478x; 1.0478x over previous
import jax
import jax.numpy as jnp
from jax.experimental import pallas as pl
from jax.experimental.pallas import tpu as pltpu


def _make_body(C, H, W, n1, n2, n3):
    HW = H * W
    s1a, p1a = 2, 6
    Ho1a = (H + 2 * p1a - 1) // s1a + 1
    Wo1a = (W + 2 * p1a - 1) // s1a + 1
    i_lo = -(-p1a // s1a)
    j_lo = -(-p1a // s1a)
    nr = (H + 1) // 2
    nc = (W + 1) // 2
    nS = nr * nc
    kh1, kw1, ph1, pw1 = 10, 10, 5, 6
    Hp1, Wp1 = Ho1a + 2 * ph1, Wo1a + 2 * pw1
    Lacc1 = (H - 1) * Wp1 + W
    N1 = 512
    Lb1 = (kh1 - 1) * Wp1 + N1
    kh2, kw2, ph2, pw2 = 3, 3, 1, 1
    Hp2, Wp2 = H + 2 * ph2, W + 2 * pw2
    Lacc2 = (H - 1) * Wp2 + W
    N2 = 384
    Lb2 = (kh2 - 1) * Wp2 + N2

    def body(x_ref, xs_ref, w12a_ref, b12a_ref, w1c_ref, b1b_ref,
             w2c_ref, b2b_ref, w3_ref, b3_ref, o_ref,
             zb1_ref, zr1_ref, zb2_ref, zr2_ref):
        f32, bf16 = jnp.float32, jnp.bfloat16
        x = x_ref[0]
        xs = xs_ref[0]

        m = jnp.max(x, axis=1, keepdims=True)
        x2 = jnp.concatenate([jnp.broadcast_to(m, (C, HW)), x], axis=0)
        x2s = jnp.concatenate([jnp.broadcast_to(m, (C, nS)), xs], axis=0)
        xin = jnp.concatenate([x2, x2s], axis=1).astype(bf16)
        b12a = b12a_ref[...]
        y12 = jnp.dot(w12a_ref[...], xin, preferred_element_type=f32) + b12a
        y12 = jnp.maximum(y12, 0.0)
        y2a = y12[n1:, :HW]
        y1as = y12[:n1, HW:]

        zb1_ref[...] = jnp.zeros_like(zb1_ref)
        rb1a = jnp.maximum(b12a[:n1], 0.0).astype(bf16)
        for r in range(Ho1a):
            st = (ph1 + r) * Wp1 + pw1
            zb1_ref[:, st:st + Wo1a] = jnp.broadcast_to(rb1a, (n1, Wo1a))
        for r in range(nr):
            st = (ph1 + i_lo + r) * Wp1 + (pw1 + j_lo)
            zb1_ref[:, st:st + nc] = y1as[:, r * nc:(r + 1) * nc].astype(bf16)

        for di in range(kh1):
            zr1_ref[di * n1:(di + 1) * n1, :] = \
                zb1_ref[:, di * Wp1:di * Wp1 + N1]

        acc1 = jnp.broadcast_to(b1b_ref[...], (n1, Lacc1)).astype(f32)
        for dj in range(kw1):
            p = jnp.dot(w1c_ref[dj], zr1_ref[...],
                        preferred_element_type=f32)
            acc1 = acc1 + p[:, dj:dj + Lacc1]
        h1 = jnp.maximum(acc1, 0.0)

        zb2_ref[...] = jnp.zeros_like(zb2_ref)
        for r in range(H):
            st = (ph2 + r) * Wp2 + pw2
            zb2_ref[:, st:st + W] = y2a[:, r * W:(r + 1) * W].astype(bf16)
        for di in range(kh2):
            zr2_ref[di * n2:(di + 1) * n2, :] = \
                zb2_ref[:, di * Wp2:di * Wp2 + N2]

        acc2 = jnp.broadcast_to(b2b_ref[...], (n2, Lacc2)).astype(f32)
        for dj in range(kw2):
            p2 = jnp.dot(w2c_ref[dj], zr2_ref[...],
                         preferred_element_type=f32)
            acc2 = acc2 + p2[:, dj:dj + Lacc2]
        h2 = jnp.maximum(acc2, 0.0)

        h1v = jnp.concatenate(
            [h1[:, r * Wp1:r * Wp1 + W] for r in range(H)], axis=1)
        h2v = jnp.concatenate(
            [h2[:, r * Wp2:r * Wp2 + W] for r in range(H)], axis=1)
        cat = jnp.concatenate([h1v, h2v], axis=0).astype(bf16)
        h3 = jnp.dot(w3_ref[...], cat, preferred_element_type=f32)
        h3 = h3 + b3_ref[...]
        o_ref[0, :n3] = jnp.maximum(h3, 0.0)
        o_ref[0, n3:] = x

    geom = dict(HW=HW, nS=nS, Lb1=Lb1, N1=N1, Lb2=Lb2, N2=N2,
                kh1=kh1, kw1=kw1, kh2=kh2, kw2=kw2)
    return body, geom


def kernel(x, w1a, b1a, w1b, b1b, w2a, b2a, w2b, b2b, w3, b3):
    B, C, H, W = x.shape
    n1, n2, n3 = w1b.shape[0], w2b.shape[0], w3.shape[0]
    body, g = _make_body(C, H, W, n1, n2, n3)
    HW, nS = g["HW"], g["nS"]
    bf16, f32 = jnp.bfloat16, jnp.float32

    x_cm = x.reshape(B, C, HW)
    xs = x[:, :, ::2, ::2].reshape(B, C, nS)
    w12a = jnp.concatenate([w1a.reshape(n1, 2 * C),
                            w2a.reshape(n2, 2 * C)], axis=0).astype(bf16)
    b12a = jnp.concatenate([b1a, b2a]).reshape(n1 + n2, 1).astype(f32)
    w1t = w1b.transpose(2, 3, 0, 1)
    w1c = jnp.stack([jnp.concatenate([w1t[di, dj] for di in range(g["kh1"])],
                                     axis=1) for dj in range(g["kw1"])]
                    ).astype(bf16)
    w2t = w2b.transpose(2, 3, 0, 1)
    w2c = jnp.stack([jnp.concatenate([w2t[di, dj] for di in range(g["kh2"])],
                                     axis=1) for dj in range(g["kw2"])]
                    ).astype(bf16)
    w3m = w3.reshape(n3, n1 + n2).astype(bf16)
    b1bm = b1b.reshape(n1, 1).astype(f32)
    b2bm = b2b.reshape(n2, 1).astype(f32)
    b3m = b3.reshape(n3, 1).astype(f32)

    out = pl.pallas_call(
        body,
        out_shape=jax.ShapeDtypeStruct((B, n3 + C, HW), f32),
        grid=(B,),
        in_specs=[
            pl.BlockSpec((1, C, HW), lambda b: (b, 0, 0)),
            pl.BlockSpec((1, C, nS), lambda b: (b, 0, 0)),
            pl.BlockSpec((n1 + n2, 2 * C), lambda b: (0, 0)),
            pl.BlockSpec((n1 + n2, 1), lambda b: (0, 0)),
            pl.BlockSpec((g["kw1"], n1, g["kh1"] * n1), lambda b: (0, 0, 0)),
            pl.BlockSpec((n1, 1), lambda b: (0, 0)),
            pl.BlockSpec((g["kw2"], n2, g["kh2"] * n2), lambda b: (0, 0, 0)),
            pl.BlockSpec((n2, 1), lambda b: (0, 0)),
            pl.BlockSpec((n3, n1 + n2), lambda b: (0, 0)),
            pl.BlockSpec((n3, 1), lambda b: (0, 0)),
        ],
        out_specs=pl.BlockSpec((1, n3 + C, HW), lambda b: (b, 0, 0)),
        scratch_shapes=[pltpu.VMEM((n1, g["Lb1"]), bf16),
                        pltpu.VMEM((g["kh1"] * n1, g["N1"]), bf16),
                        pltpu.VMEM((n2, g["Lb2"]), bf16),
                        pltpu.VMEM((g["kh2"] * n2, g["N2"]), bf16)],
        compiler_params=pltpu.CompilerParams(dimension_semantics=("parallel",)),
    )(x_cm, xs, w12a, b12a, w1c, b1bm, w2c, b2bm, w3m, b3m)
    return out.reshape(B, n3 + C, H, W)


# speedup vs baseline: 1.0842x; 1.0347x over previous
import jax
import jax.numpy as jnp
from jax.experimental import pallas as pl
from jax.experimental.pallas import tpu as pltpu


def _make_body(C, H, W, n1, n2, n3):
    HW = H * W
    s1a, p1a = 2, 6
    Ho1a = (H + 2 * p1a - 1) // s1a + 1
    Wo1a = (W + 2 * p1a - 1) // s1a + 1
    i_lo = -(-p1a // s1a)
    j_lo = -(-p1a // s1a)
    nr = (H + 1) // 2
    nc = (W + 1) // 2
    nS = nr * nc
    kh1, kw1, ph1, pw1 = 10, 10, 5, 6
    Hp1, Wp1 = Ho1a + 2 * ph1, Wo1a + 2 * pw1
    Lacc1 = (H - 1) * Wp1 + W
    N1 = 512
    Lb1 = (kh1 - 1) * Wp1 + N1
    kh2, kw2, ph2, pw2 = 3, 3, 1, 1
    Hp2, Wp2 = H + 2 * ph2, W + 2 * pw2
    Lacc2 = (H - 1) * Wp2 + W
    N2 = 384
    Lb2 = (kh2 - 1) * Wp2 + N2

    G = 2

    def body(x_ref, xs_ref, w12a_ref, b12a_ref, w1c_ref, b1b_ref,
             w2c_ref, b2b_ref, w3_ref, b3_ref, o_ref,
             zb1_ref, zr1_ref, zb2_ref, zr2_ref):
      f32, bf16 = jnp.float32, jnp.bfloat16
      for gi in range(G):
        x = x_ref[gi]
        xs = xs_ref[gi]
        zb1 = zb1_ref.at[gi * n1:(gi + 1) * n1]
        zr1 = zr1_ref.at[gi * kh1 * n1:(gi + 1) * kh1 * n1]
        zb2 = zb2_ref.at[gi * n2:(gi + 1) * n2]
        zr2 = zr2_ref.at[gi * kh2 * n2:(gi + 1) * kh2 * n2]

        m = jnp.max(x, axis=1, keepdims=True)
        x2 = jnp.concatenate([jnp.broadcast_to(m, (C, HW)), x], axis=0)
        x2s = jnp.concatenate([jnp.broadcast_to(m, (C, nS)), xs], axis=0)
        xin = jnp.concatenate([x2, x2s], axis=1).astype(bf16)
        b12a = b12a_ref[...]
        y12 = jnp.dot(w12a_ref[...], xin, preferred_element_type=f32) + b12a
        y12 = jnp.maximum(y12, 0.0)
        y2a = y12[n1:, :HW]
        y1as = y12[:n1, HW:]

        zb1[...] = jnp.zeros((n1, Lb1), jnp.bfloat16)
        rb1a = jnp.maximum(b12a[:n1], 0.0).astype(bf16)
        for r in range(Ho1a):
            st = (ph1 + r) * Wp1 + pw1
            zb1[:, st:st + Wo1a] = jnp.broadcast_to(rb1a, (n1, Wo1a))
        for r in range(nr):
            st = (ph1 + i_lo + r) * Wp1 + (pw1 + j_lo)
            zb1[:, st:st + nc] = y1as[:, r * nc:(r + 1) * nc].astype(bf16)

        for di in range(kh1):
            zr1[di * n1:(di + 1) * n1, :] = \
                zb1[:, di * Wp1:di * Wp1 + N1]

        acc1 = jnp.broadcast_to(b1b_ref[...], (n1, Lacc1)).astype(f32)
        for dj in range(kw1):
            p = jnp.dot(w1c_ref[dj], zr1[...],
                        preferred_element_type=f32)
            acc1 = acc1 + p[:, dj:dj + Lacc1]
        h1 = jnp.maximum(acc1, 0.0)

        zb2[...] = jnp.zeros((n2, Lb2), jnp.bfloat16)
        for r in range(H):
            st = (ph2 + r) * Wp2 + pw2
            zb2[:, st:st + W] = y2a[:, r * W:(r + 1) * W].astype(bf16)
        for di in range(kh2):
            zr2[di * n2:(di + 1) * n2, :] = \
                zb2[:, di * Wp2:di * Wp2 + N2]

        acc2 = jnp.broadcast_to(b2b_ref[...], (n2, Lacc2)).astype(f32)
        for dj in range(kw2):
            p2 = jnp.dot(w2c_ref[dj], zr2[...],
                         preferred_element_type=f32)
            acc2 = acc2 + p2[:, dj:dj + Lacc2]
        h2 = jnp.maximum(acc2, 0.0)

        h1v = jnp.concatenate(
            [h1[:, r * Wp1:r * Wp1 + W] for r in range(H)], axis=1)
        h2v = jnp.concatenate(
            [h2[:, r * Wp2:r * Wp2 + W] for r in range(H)], axis=1)
        cat = jnp.concatenate([h1v, h2v], axis=0).astype(bf16)
        h3 = jnp.dot(w3_ref[...], cat, preferred_element_type=f32)
        h3 = h3 + b3_ref[...]
        o_ref[gi, :n3] = jnp.maximum(h3, 0.0)
        o_ref[gi, n3:] = x

    geom = dict(HW=HW, nS=nS, Lb1=Lb1, N1=N1, Lb2=Lb2, N2=N2,
                kh1=kh1, kw1=kw1, kh2=kh2, kw2=kw2, G=G)
    return body, geom


def kernel(x, w1a, b1a, w1b, b1b, w2a, b2a, w2b, b2b, w3, b3):
    B, C, H, W = x.shape
    n1, n2, n3 = w1b.shape[0], w2b.shape[0], w3.shape[0]
    body, g = _make_body(C, H, W, n1, n2, n3)
    HW, nS = g["HW"], g["nS"]
    bf16, f32 = jnp.bfloat16, jnp.float32

    x_cm = x.reshape(B, C, HW)
    xs = x[:, :, ::2, ::2].reshape(B, C, nS)
    w12a = jnp.concatenate([w1a.reshape(n1, 2 * C),
                            w2a.reshape(n2, 2 * C)], axis=0).astype(bf16)
    b12a = jnp.concatenate([b1a, b2a]).reshape(n1 + n2, 1).astype(f32)
    w1t = w1b.transpose(2, 3, 0, 1)
    w1c = jnp.stack([jnp.concatenate([w1t[di, dj] for di in range(g["kh1"])],
                                     axis=1) for dj in range(g["kw1"])]
                    ).astype(bf16)
    w2t = w2b.transpose(2, 3, 0, 1)
    w2c = jnp.stack([jnp.concatenate([w2t[di, dj] for di in range(g["kh2"])],
                                     axis=1) for dj in range(g["kw2"])]
                    ).astype(bf16)
    w3m = w3.reshape(n3, n1 + n2).astype(bf16)
    b1bm = b1b.reshape(n1, 1).astype(f32)
    b2bm = b2b.reshape(n2, 1).astype(f32)
    b3m = b3.reshape(n3, 1).astype(f32)

    out = pl.pallas_call(
        body,
        out_shape=jax.ShapeDtypeStruct((B, n3 + C, HW), f32),
        grid=(B // g["G"],),
        in_specs=[
            pl.BlockSpec((g["G"], C, HW), lambda b: (b, 0, 0)),
            pl.BlockSpec((g["G"], C, nS), lambda b: (b, 0, 0)),
            pl.BlockSpec((n1 + n2, 2 * C), lambda b: (0, 0)),
            pl.BlockSpec((n1 + n2, 1), lambda b: (0, 0)),
            pl.BlockSpec((g["kw1"], n1, g["kh1"] * n1), lambda b: (0, 0, 0)),
            pl.BlockSpec((n1, 1), lambda b: (0, 0)),
            pl.BlockSpec((g["kw2"], n2, g["kh2"] * n2), lambda b: (0, 0, 0)),
            pl.BlockSpec((n2, 1), lambda b: (0, 0)),
            pl.BlockSpec((n3, n1 + n2), lambda b: (0, 0)),
            pl.BlockSpec((n3, 1), lambda b: (0, 0)),
        ],
        out_specs=pl.BlockSpec((g["G"], n3 + C, HW), lambda b: (b, 0, 0)),
        scratch_shapes=[pltpu.VMEM((g["G"] * n1, g["Lb1"]), bf16),
                        pltpu.VMEM((g["G"] * g["kh1"] * n1, g["N1"]), bf16),
                        pltpu.VMEM((g["G"] * n2, g["Lb2"]), bf16),
                        pltpu.VMEM((g["G"] * g["kh2"] * n2, g["N2"]), bf16)],
        compiler_params=pltpu.CompilerParams(dimension_semantics=("parallel",)),
    )(x_cm, xs, w12a, b12a, w1c, b1bm, w2c, b2bm, w3m, b3m)
    return out.reshape(B, n3 + C, H, W)


# speedup vs baseline: 1.1070x; 1.0211x over previous
import jax
import jax.numpy as jnp
from jax.experimental import pallas as pl
from jax.experimental.pallas import tpu as pltpu


def _make_body(C, H, W, n1, n2, n3):
    HW = H * W
    s1a, p1a = 2, 6
    Ho1a = (H + 2 * p1a - 1) // s1a + 1
    Wo1a = (W + 2 * p1a - 1) // s1a + 1
    i_lo = -(-p1a // s1a)
    j_lo = -(-p1a // s1a)
    nr = (H + 1) // 2
    nc = (W + 1) // 2
    nS = nr * nc
    kh1, kw1, ph1, pw1 = 10, 10, 5, 6
    Hp1, Wp1 = Ho1a + 2 * ph1, Wo1a + 2 * pw1
    Lacc1 = (H - 1) * Wp1 + W
    N1 = 512
    Lb1 = (kh1 - 1) * Wp1 + N1
    kh2, kw2, ph2, pw2 = 3, 3, 1, 1
    Hp2, Wp2 = H + 2 * ph2, W + 2 * pw2
    Lacc2 = (H - 1) * Wp2 + W
    N2 = 384
    Lb2 = (kh2 - 1) * Wp2 + N2

    G = 4

    def body(x_ref, xs_ref, w12a_ref, b12a_ref, w1c_ref, b1b_ref,
             w2c_ref, b2b_ref, w3_ref, b3_ref, o_ref,
             zb1_ref, zr1_ref, zb2_ref, zr2_ref):
      f32, bf16 = jnp.float32, jnp.bfloat16
      for gi in range(G):
        x = x_ref[gi]
        xs = xs_ref[gi]
        zb1 = zb1_ref.at[gi * n1:(gi + 1) * n1]
        zr1 = zr1_ref.at[gi * kh1 * n1:(gi + 1) * kh1 * n1]
        zb2 = zb2_ref.at[gi * n2:(gi + 1) * n2]
        zr2 = zr2_ref.at[gi * kh2 * n2:(gi + 1) * kh2 * n2]

        m = jnp.max(x, axis=1, keepdims=True)
        x2 = jnp.concatenate([jnp.broadcast_to(m, (C, HW)), x], axis=0)
        x2s = jnp.concatenate([jnp.broadcast_to(m, (C, nS)), xs], axis=0)
        xin = jnp.concatenate([x2, x2s], axis=1).astype(bf16)
        b12a = b12a_ref[...]
        y12 = jnp.dot(w12a_ref[...], xin, preferred_element_type=f32) + b12a
        y12 = jnp.maximum(y12, 0.0)
        y2a = y12[n1:, :HW]
        y1as = y12[:n1, HW:]

        zb1[...] = jnp.zeros((n1, Lb1), jnp.bfloat16)
        rb1a = jnp.maximum(b12a[:n1], 0.0).astype(bf16)
        for r in range(Ho1a):
            st = (ph1 + r) * Wp1 + pw1
            zb1[:, st:st + Wo1a] = jnp.broadcast_to(rb1a, (n1, Wo1a))
        for r in range(nr):
            st = (ph1 + i_lo + r) * Wp1 + (pw1 + j_lo)
            zb1[:, st:st + nc] = y1as[:, r * nc:(r + 1) * nc].astype(bf16)

        for di in range(kh1):
            zr1[di * n1:(di + 1) * n1, :] = \
                zb1[:, di * Wp1:di * Wp1 + N1]

        acc1 = jnp.broadcast_to(b1b_ref[...], (n1, Lacc1)).astype(f32)
        for dj in range(kw1):
            p = jnp.dot(w1c_ref[dj], zr1[...],
                        preferred_element_type=f32)
            acc1 = acc1 + p[:, dj:dj + Lacc1]
        h1 = jnp.maximum(acc1, 0.0)

        zb2[...] = jnp.zeros((n2, Lb2), jnp.bfloat16)
        for r in range(H):
            st = (ph2 + r) * Wp2 + pw2
            zb2[:, st:st + W] = y2a[:, r * W:(r + 1) * W].astype(bf16)
        for di in range(kh2):
            zr2[di * n2:(di + 1) * n2, :] = \
                zb2[:, di * Wp2:di * Wp2 + N2]

        acc2 = jnp.broadcast_to(b2b_ref[...], (n2, Lacc2)).astype(f32)
        for dj in range(kw2):
            p2 = jnp.dot(w2c_ref[dj], zr2[...],
                         preferred_element_type=f32)
            acc2 = acc2 + p2[:, dj:dj + Lacc2]
        h2 = jnp.maximum(acc2, 0.0)

        h1v = jnp.concatenate(
            [h1[:, r * Wp1:r * Wp1 + W] for r in range(H)], axis=1)
        h2v = jnp.concatenate(
            [h2[:, r * Wp2:r * Wp2 + W] for r in range(H)], axis=1)
        cat = jnp.concatenate([h1v, h2v], axis=0).astype(bf16)
        h3 = jnp.dot(w3_ref[...], cat, preferred_element_type=f32)
        h3 = h3 + b3_ref[...]
        o_ref[gi, :n3] = jnp.maximum(h3, 0.0)
        o_ref[gi, n3:] = x

    geom = dict(HW=HW, nS=nS, Lb1=Lb1, N1=N1, Lb2=Lb2, N2=N2,
                kh1=kh1, kw1=kw1, kh2=kh2, kw2=kw2, G=G)
    return body, geom


def kernel(x, w1a, b1a, w1b, b1b, w2a, b2a, w2b, b2b, w3, b3):
    B, C, H, W = x.shape
    n1, n2, n3 = w1b.shape[0], w2b.shape[0], w3.shape[0]
    body, g = _make_body(C, H, W, n1, n2, n3)
    HW, nS = g["HW"], g["nS"]
    bf16, f32 = jnp.bfloat16, jnp.float32

    x_cm = x.reshape(B, C, HW)
    xs = x[:, :, ::2, ::2].reshape(B, C, nS)
    w12a = jnp.concatenate([w1a.reshape(n1, 2 * C),
                            w2a.reshape(n2, 2 * C)], axis=0).astype(bf16)
    b12a = jnp.concatenate([b1a, b2a]).reshape(n1 + n2, 1).astype(f32)
    w1t = w1b.transpose(2, 3, 0, 1)
    w1c = jnp.stack([jnp.concatenate([w1t[di, dj] for di in range(g["kh1"])],
                                     axis=1) for dj in range(g["kw1"])]
                    ).astype(bf16)
    w2t = w2b.transpose(2, 3, 0, 1)
    w2c = jnp.stack([jnp.concatenate([w2t[di, dj] for di in range(g["kh2"])],
                                     axis=1) for dj in range(g["kw2"])]
                    ).astype(bf16)
    w3m = w3.reshape(n3, n1 + n2).astype(bf16)
    b1bm = b1b.reshape(n1, 1).astype(f32)
    b2bm = b2b.reshape(n2, 1).astype(f32)
    b3m = b3.reshape(n3, 1).astype(f32)

    out = pl.pallas_call(
        body,
        out_shape=jax.ShapeDtypeStruct((B, n3 + C, HW), f32),
        grid=(B // g["G"],),
        in_specs=[
            pl.BlockSpec((g["G"], C, HW), lambda b: (b, 0, 0)),
            pl.BlockSpec((g["G"], C, nS), lambda b: (b, 0, 0)),
            pl.BlockSpec((n1 + n2, 2 * C), lambda b: (0, 0)),
            pl.BlockSpec((n1 + n2, 1), lambda b: (0, 0)),
            pl.BlockSpec((g["kw1"], n1, g["kh1"] * n1), lambda b: (0, 0, 0)),
            pl.BlockSpec((n1, 1), lambda b: (0, 0)),
            pl.BlockSpec((g["kw2"], n2, g["kh2"] * n2), lambda b: (0, 0, 0)),
            pl.BlockSpec((n2, 1), lambda b: (0, 0)),
            pl.BlockSpec((n3, n1 + n2), lambda b: (0, 0)),
            pl.BlockSpec((n3, 1), lambda b: (0, 0)),
        ],
        out_specs=pl.BlockSpec((g["G"], n3 + C, HW), lambda b: (b, 0, 0)),
        scratch_shapes=[pltpu.VMEM((g["G"] * n1, g["Lb1"]), bf16),
                        pltpu.VMEM((g["G"] * g["kh1"] * n1, g["N1"]), bf16),
                        pltpu.VMEM((g["G"] * n2, g["Lb2"]), bf16),
                        pltpu.VMEM((g["G"] * g["kh2"] * n2, g["N2"]), bf16)],
        compiler_params=pltpu.CompilerParams(dimension_semantics=("parallel",)),
    )(x_cm, xs, w12a, b12a, w1c, b1bm, w2c, b2bm, w3m, b3m)
    return out.reshape(B, n3 + C, H, W)


# speedup vs baseline: 1.1152x; 1.0074x over previous
import jax
import jax.numpy as jnp
from jax.experimental import pallas as pl
from jax.experimental.pallas import tpu as pltpu


def _make_body(C, H, W, n1, n2, n3):
    HW = H * W
    s1a, p1a = 2, 6
    Ho1a = (H + 2 * p1a - 1) // s1a + 1
    Wo1a = (W + 2 * p1a - 1) // s1a + 1
    i_lo = -(-p1a // s1a)
    j_lo = -(-p1a // s1a)
    nr = (H + 1) // 2
    nc = (W + 1) // 2
    nS = nr * nc
    kh1, kw1, ph1, pw1 = 10, 10, 5, 6
    Hp1, Wp1 = Ho1a + 2 * ph1, Wo1a + 2 * pw1
    Lacc1 = (H - 1) * Wp1 + W
    N1 = 512
    Lb1 = (kh1 - 1) * Wp1 + N1
    kh2, kw2, ph2, pw2 = 3, 3, 1, 1
    Hp2, Wp2 = H + 2 * ph2, W + 2 * pw2
    Lacc2 = (H - 1) * Wp2 + W
    N2 = 384
    Lb2 = (kh2 - 1) * Wp2 + N2

    G = 8

    def body(x_ref, xs_ref, w12a_ref, b12a_ref, w1c_ref, b1b_ref,
             w2c_ref, b2b_ref, w3_ref, b3_ref, o_ref,
             zb1_ref, zr1_ref, zb2_ref, zr2_ref):
      f32, bf16 = jnp.float32, jnp.bfloat16
      for gi in range(G):
        x = x_ref[gi]
        xs = xs_ref[gi]
        zb1 = zb1_ref.at[gi * n1:(gi + 1) * n1]
        zr1 = zr1_ref.at[gi * kh1 * n1:(gi + 1) * kh1 * n1]
        zb2 = zb2_ref.at[gi * n2:(gi + 1) * n2]
        zr2 = zr2_ref.at[gi * kh2 * n2:(gi + 1) * kh2 * n2]

        m = jnp.max(x, axis=1, keepdims=True)
        x2 = jnp.concatenate([jnp.broadcast_to(m, (C, HW)), x], axis=0)
        x2s = jnp.concatenate([jnp.broadcast_to(m, (C, nS)), xs], axis=0)
        xin = jnp.concatenate([x2, x2s], axis=1).astype(bf16)
        b12a = b12a_ref[...]
        y12 = jnp.dot(w12a_ref[...], xin, preferred_element_type=f32) + b12a
        y12 = jnp.maximum(y12, 0.0)
        y2a = y12[n1:, :HW]
        y1as = y12[:n1, HW:]

        zb1[...] = jnp.zeros((n1, Lb1), jnp.bfloat16)
        rb1a = jnp.maximum(b12a[:n1], 0.0).astype(bf16)
        for r in range(Ho1a):
            st = (ph1 + r) * Wp1 + pw1
            zb1[:, st:st + Wo1a] = jnp.broadcast_to(rb1a, (n1, Wo1a))
        for r in range(nr):
            st = (ph1 + i_lo + r) * Wp1 + (pw1 + j_lo)
            zb1[:, st:st + nc] = y1as[:, r * nc:(r + 1) * nc].astype(bf16)

        for di in range(kh1):
            zr1[di * n1:(di + 1) * n1, :] = \
                zb1[:, di * Wp1:di * Wp1 + N1]

        acc1 = jnp.broadcast_to(b1b_ref[...], (n1, Lacc1)).astype(f32)
        for dj in range(kw1):
            p = jnp.dot(w1c_ref[dj], zr1[...],
                        preferred_element_type=f32)
            acc1 = acc1 + p[:, dj:dj + Lacc1]
        h1 = jnp.maximum(acc1, 0.0)

        zb2[...] = jnp.zeros((n2, Lb2), jnp.bfloat16)
        for r in range(H):
            st = (ph2 + r) * Wp2 + pw2
            zb2[:, st:st + W] = y2a[:, r * W:(r + 1) * W].astype(bf16)
        for di in range(kh2):
            zr2[di * n2:(di + 1) * n2, :] = \
                zb2[:, di * Wp2:di * Wp2 + N2]

        acc2 = jnp.broadcast_to(b2b_ref[...], (n2, Lacc2)).astype(f32)
        for dj in range(kw2):
            p2 = jnp.dot(w2c_ref[dj], zr2[...],
                         preferred_element_type=f32)
            acc2 = acc2 + p2[:, dj:dj + Lacc2]
        h2 = jnp.maximum(acc2, 0.0)

        h1v = jnp.concatenate(
            [h1[:, r * Wp1:r * Wp1 + W] for r in range(H)], axis=1)
        h2v = jnp.concatenate(
            [h2[:, r * Wp2:r * Wp2 + W] for r in range(H)], axis=1)
        cat = jnp.concatenate([h1v, h2v], axis=0).astype(bf16)
        h3 = jnp.dot(w3_ref[...], cat, preferred_element_type=f32)
        h3 = h3 + b3_ref[...]
        o_ref[gi, :n3] = jnp.maximum(h3, 0.0)
        o_ref[gi, n3:] = x

    geom = dict(HW=HW, nS=nS, Lb1=Lb1, N1=N1, Lb2=Lb2, N2=N2,
                kh1=kh1, kw1=kw1, kh2=kh2, kw2=kw2, G=G)
    return body, geom


def kernel(x, w1a, b1a, w1b, b1b, w2a, b2a, w2b, b2b, w3, b3):
    B, C, H, W = x.shape
    n1, n2, n3 = w1b.shape[0], w2b.shape[0], w3.shape[0]
    body, g = _make_body(C, H, W, n1, n2, n3)
    HW, nS = g["HW"], g["nS"]
    bf16, f32 = jnp.bfloat16, jnp.float32

    x_cm = x.reshape(B, C, HW)
    xs = x[:, :, ::2, ::2].reshape(B, C, nS)
    w12a = jnp.concatenate([w1a.reshape(n1, 2 * C),
                            w2a.reshape(n2, 2 * C)], axis=0).astype(bf16)
    b12a = jnp.concatenate([b1a, b2a]).reshape(n1 + n2, 1).astype(f32)
    w1t = w1b.transpose(2, 3, 0, 1)
    w1c = jnp.stack([jnp.concatenate([w1t[di, dj] for di in range(g["kh1"])],
                                     axis=1) for dj in range(g["kw1"])]
                    ).astype(bf16)
    w2t = w2b.transpose(2, 3, 0, 1)
    w2c = jnp.stack([jnp.concatenate([w2t[di, dj] for di in range(g["kh2"])],
                                     axis=1) for dj in range(g["kw2"])]
                    ).astype(bf16)
    w3m = w3.reshape(n3, n1 + n2).astype(bf16)
    b1bm = b1b.reshape(n1, 1).astype(f32)
    b2bm = b2b.reshape(n2, 1).astype(f32)
    b3m = b3.reshape(n3, 1).astype(f32)

    out = pl.pallas_call(
        body,
        out_shape=jax.ShapeDtypeStruct((B, n3 + C, HW), f32),
        grid=(B // g["G"],),
        in_specs=[
            pl.BlockSpec((g["G"], C, HW), lambda b: (b, 0, 0)),
            pl.BlockSpec((g["G"], C, nS), lambda b: (b, 0, 0)),
            pl.BlockSpec((n1 + n2, 2 * C), lambda b: (0, 0)),
            pl.BlockSpec((n1 + n2, 1), lambda b: (0, 0)),
            pl.BlockSpec((g["kw1"], n1, g["kh1"] * n1), lambda b: (0, 0, 0)),
            pl.BlockSpec((n1, 1), lambda b: (0, 0)),
            pl.BlockSpec((g["kw2"], n2, g["kh2"] * n2), lambda b: (0, 0, 0)),
            pl.BlockSpec((n2, 1), lambda b: (0, 0)),
            pl.BlockSpec((n3, n1 + n2), lambda b: (0, 0)),
            pl.BlockSpec((n3, 1), lambda b: (0, 0)),
        ],
        out_specs=pl.BlockSpec((g["G"], n3 + C, HW), lambda b: (b, 0, 0)),
        scratch_shapes=[pltpu.VMEM((g["G"] * n1, g["Lb1"]), bf16),
                        pltpu.VMEM((g["G"] * g["kh1"] * n1, g["N1"]), bf16),
                        pltpu.VMEM((g["G"] * n2, g["Lb2"]), bf16),
                        pltpu.VMEM((g["G"] * g["kh2"] * n2, g["N2"]), bf16)],
        compiler_params=pltpu.CompilerParams(dimension_semantics=("parallel",)),
    )(x_cm, xs, w12a, b12a, w1c, b1bm, w2c, b2bm, w3m, b3m)
    return out.reshape(B, n3 + C, H, W)
